# Initial kernel scaffold; baseline (speedup 1.0000x reference)
import jax
import jax.numpy as jnp
from jax.experimental import pallas as pl


def kernel(subs, rels, edges, nodes, old_idx, params):
    raise NotImplementedError("write your pallas kernel here")


# speedup vs baseline: 1.6221x; 1.6221x over previous
import functools

import jax
import jax.numpy as jnp
from jax import lax
from jax.experimental import pallas as pl
from jax.experimental.pallas import tpu as pltpu
from jax.experimental.pallas import tpu_sc as plsc

HID = 128
ATT = 5
NVOC = 475
NNODE = 10000
NQ = 16
NE = 160000
NL = 2
NXL = 4

NC = 2
NS = 16
NW = NC * NS
CH = 128
K = 40
NEP = NW * K * CH
DM = 160
DMH = 80
NSEG = 10016
BE = 1280
BN = 2000

@functools.cache
def _mesh():
    return plsc.VectorSubcoreMesh(core_axis_name="c", subcore_axis_name="s",
                                  num_cores=NC, num_subcores=NS)


_SC_PARAMS = pltpu.CompilerParams(use_tc_tiling_on_sc=False)


def _wid():
    return lax.axis_index("s") * NC + lax.axis_index("c")


def _gather2_body(tab1, tab2, idx1, idx2, out1, out2,
                  idxb, rows, gs0, gs1, ss0, ss1):
    w = _wid()

    def run_table(tab, idx_hbm, out):
        pltpu.sync_copy(idx_hbm.at[w], idxb)
        base = w * (K * CH)

        def do_pair(i, _):
            j0 = 2 * i
            j1 = j0 + 1
            g0 = pltpu.make_async_copy(tab.at[idxb.at[j0]], rows.at[0], gs0)
            g1 = pltpu.make_async_copy(tab.at[idxb.at[j1]], rows.at[1], gs1)
            g0.start()
            g1.start()
            s0 = pltpu.make_async_copy(rows.at[0],
                                       out.at[pl.ds(base + j0 * CH, CH)], ss0)
            s1 = pltpu.make_async_copy(rows.at[1],
                                       out.at[pl.ds(base + j1 * CH, CH)], ss1)
            g0.wait()
            s0.start()
            g1.wait()
            s1.start()
            s0.wait()
            s1.wait()
            return 0

        lax.fori_loop(0, K // 2, do_pair, 0)

    run_table(tab1, idx1, out1)
    run_table(tab2, idx2, out2)


@functools.partial(jax.jit, static_argnames=())
def _sc_gather2(tab1, tab2, idx1, idx2):
    f = pl.kernel(
        _gather2_body,
        out_type=[jax.ShapeDtypeStruct((NEP, HID), jnp.float32),
                  jax.ShapeDtypeStruct((NEP, HID), jnp.float32)],
        mesh=_mesh(),
        scratch_types=[pltpu.VMEM((K, CH), jnp.int32),
                       pltpu.VMEM((2, CH, HID), jnp.float32),
                       pltpu.SemaphoreType.DMA,
                       pltpu.SemaphoreType.DMA,
                       pltpu.SemaphoreType.DMA,
                       pltpu.SemaphoreType.DMA],
        compiler_params=_SC_PARAMS,
    )
    return f(tab1, tab2, idx1, idx2)


def _scatter_body(msgx, obj_idx, zrows, out, idxb, rows, ls0, ls1, shared):
    c = lax.axis_index("c")
    s = lax.axis_index("s")
    seg_per_tile = NSEG // NS
    t0 = s * seg_per_tile
    col0 = c * DMH

    pltpu.sync_copy(zrows, shared.at[pl.ds(t0, seg_per_tile)])
    plsc.subcore_barrier()

    def run_span(w):
        pltpu.sync_copy(obj_idx.at[w], idxb)
        base = w * (K * CH)

        def do_pair(i, _):
            j0 = 2 * i
            j1 = j0 + 1
            l0 = pltpu.make_async_copy(
                msgx.at[pl.ds(base + j0 * CH, CH), pl.ds(col0, DMH)],
                rows.at[0], ls0)
            l1 = pltpu.make_async_copy(
                msgx.at[pl.ds(base + j1 * CH, CH), pl.ds(col0, DMH)],
                rows.at[1], ls1)
            l0.start()
            l1.start()
            l0.wait()
            pltpu.sync_copy(rows.at[0], shared.at[idxb.at[j0]], add=True)
            l1.wait()
            pltpu.sync_copy(rows.at[1], shared.at[idxb.at[j1]], add=True)
            return 0

        lax.fori_loop(0, K // 2, do_pair, 0)

    run_span(2 * s)
    run_span(2 * s + 1)
    plsc.subcore_barrier()
    pltpu.sync_copy(shared.at[pl.ds(t0, seg_per_tile)],
                    out.at[pl.ds(t0, seg_per_tile), pl.ds(col0, DMH)])


@jax.jit
def _sc_scatter(msgx, obj_idx, zrows):
    f = pl.kernel(
        _scatter_body,
        out_type=jax.ShapeDtypeStruct((NSEG, DM), jnp.float32),
        mesh=_mesh(),
        scratch_types=[pltpu.VMEM((K, CH), jnp.int32),
                       pltpu.VMEM((2, CH, DMH), jnp.float32),
                       pltpu.SemaphoreType.DMA,
                       pltpu.SemaphoreType.DMA,
                       pltpu.VMEM_SHARED((NSEG, DMH), jnp.float32)],
        compiler_params=_SC_PARAMS,
    )
    return f(msgx, obj_idx, zrows)


def _edge_body(hs_ref, hr_ref, r_ref, qre_ref, gW1_ref, gb1_ref, gW2_ref,
               gb2_ref, Ws_ref, Wqr_ref, bqr_ref, wa_ref, ba_ref, out_ref):
    hs = hs_ref[...]
    hr = hr_ref[...]
    r = r_ref[0]
    iot = lax.broadcasted_iota(jnp.int32, (NQ, BE), 0)
    oh = (iot == r).astype(jnp.float32)
    qre = qre_ref[...]
    dn = (((0,), (0,)), ((), ()))
    hqr = lax.dot_general(oh, qre, dn, preferred_element_type=jnp.float32)
    gW1 = gW1_ref[...]
    g = hr @ gW1[:HID] + hqr @ gW1[HID:2 * HID] + hs @ gW1[2 * HID:]
    g = 1.0 / (1.0 + jnp.exp(-(g + gb1_ref[...])))
    u = g[:, :HID]
    rs = g[:, HID:]
    gW2 = gW2_ref[...]
    cand = jnp.tanh(hr @ gW2[:HID] + (rs * hs) @ gW2[HID:] + gb2_ref[...])
    uri = (1.0 - u) * hs + u * cand
    qa = qre @ Wqr_ref[...] + bqr_ref[...]
    s = jnp.maximum(uri @ Ws_ref[...] +
                    lax.dot_general(oh, qa, dn,
                                    preferred_element_type=jnp.float32), 0.0)
    aw = s @ wa_ref[...] + ba_ref[...]
    ue = jnp.exp(aw)
    out_ref[...] = jnp.concatenate(
        [ue * uri, ue, jnp.zeros((BE, DM - HID - 1), jnp.float32)], axis=1)


def _full(shape):
    nd = len(shape)
    return pl.BlockSpec(shape, lambda i, _n=nd: (0,) * _n)


@jax.jit
def _tc_edge(hs, hr, r3, qre, gW1, gb1, gW2, gb2, Ws, Wqr, bqr, wa, ba):
    nblk = NEP // BE
    return pl.pallas_call(
        _edge_body,
        out_shape=jax.ShapeDtypeStruct((NEP, DM), jnp.float32),
        grid=(nblk,),
        in_specs=[
            pl.BlockSpec((BE, HID), lambda i: (i, 0)),
            pl.BlockSpec((BE, HID), lambda i: (i, 0)),
            pl.BlockSpec((1, 1, BE), lambda i: (i, 0, 0)),
            _full((NQ, HID)), _full((3 * HID, 2 * HID)), _full((1, 2 * HID)),
            _full((2 * HID, HID)), _full((1, HID)), _full((HID, ATT)),
            _full((HID, ATT)), _full((1, ATT)), _full((ATT, 1)),
            _full((1, 1)),
        ],
        out_specs=pl.BlockSpec((BE, DM), lambda i: (i, 0)),
    )(hs, hr, r3, qre, gW1, gb1, gW2, gb2, Ws, Wqr, bqr, wa, ba)


def _node_body(a0_ref, ng_ref, h0_ref, qre_ref, Wh_ref, gW1_ref,
               gb1_ref, gW2_ref, gb2_ref, Wf_ref, hid_ref, sc_ref):
    a = a0_ref[...]
    agg = a[:, :HID]
    se = a[:, HID:HID + 1]
    m = jnp.maximum((agg / se) @ Wh_ref[...], 0.0)
    ng = ng_ref[0]
    iot = lax.broadcasted_iota(jnp.int32, (NQ, BN), 0)
    oh = (iot == ng).astype(jnp.float32)
    dn = (((0,), (0,)), ((), ()))
    hqr = lax.dot_general(oh, qre_ref[...], dn,
                          preferred_element_type=jnp.float32)
    h = h0_ref[...]
    gW1 = gW1_ref[...]
    g = m @ gW1[:HID] + hqr @ gW1[HID:2 * HID] + h @ gW1[2 * HID:]
    g = 1.0 / (1.0 + jnp.exp(-(g + gb1_ref[...])))
    u = g[:, :HID]
    rs = g[:, HID:]
    gW2 = gW2_ref[...]
    cand = jnp.tanh(m @ gW2[:HID] + (rs * h) @ gW2[HID:] + gb2_ref[...])
    out = (1.0 - u) * h + u * cand
    hid_ref[...] = out
    sc_ref[...] = out @ Wf_ref[...]


@jax.jit
def _tc_node(a0, ng3, h0, qre, Wh, gW1, gb1, gW2, gb2, Wf):
    nblk = NNODE // BN
    return pl.pallas_call(
        _node_body,
        out_shape=[jax.ShapeDtypeStruct((NNODE, HID), jnp.float32),
                   jax.ShapeDtypeStruct((NNODE, 1), jnp.float32)],
        grid=(nblk,),
        in_specs=[
            pl.BlockSpec((BN, DM), lambda i: (i, 0)),
            pl.BlockSpec((1, 1, BN), lambda i: (i, 0, 0)),
            pl.BlockSpec((BN, HID), lambda i: (i, 0)),
            _full((NQ, HID)), _full((HID, HID)), _full((3 * HID, 2 * HID)),
            _full((1, 2 * HID)), _full((2 * HID, HID)), _full((1, HID)),
            _full((HID, 1)),
        ],
        out_specs=[pl.BlockSpec((BN, HID), lambda i: (i, 0)),
                   pl.BlockSpec((BN, 1), lambda i: (i, 0))],
    )(a0, ng3, h0, qre, Wh, gW1, gb1, gW2, gb2, Wf)


def kernel(subs, rels, edges, nodes, old_idx, params):
    n = subs.shape[0]
    idt = edges.dtype
    r_idx = edges[:, 0]
    rel = edges[:, 2]
    sub = edges[:, 4]
    obj = edges[:, 5]
    sub2 = old_idx[sub]

    node_group = jnp.zeros((NNODE,), dtype=idt).at[obj].set(r_idx)

    pad = NEP - NE
    def p32(x, fill):
        return jnp.concatenate(
            [x, jnp.full((pad,), fill, x.dtype)]).reshape(NW, K, CH)

    sub_sc = p32(sub, 0)
    sub2_sc = p32(sub2, 0)
    rel_sc = p32(rel, 0)
    obj_sc = p32(obj, NNODE)
    r3 = jnp.concatenate([r_idx, jnp.zeros((pad,), idt)]).reshape(
        NEP // BE, 1, BE)
    ng3 = node_group.reshape(NNODE // BN, 1, BN)
    zrows = jnp.zeros((NSEG // NS, DMH), jnp.float32)

    layers = params["layers"]
    top = params["top"]
    tp = [top["gW1"], top["gb1"].reshape(1, -1),
          top["gW2"], top["gb2"].reshape(1, -1)]
    Wf = params["Wfinal"]

    def run_layer(p, hidden, h0_in, sub_idx):
        qre = p["rela"][rels]
        hs, hr = _sc_gather2(hidden, p["rela"], sub_idx, rel_sc)
        msgx = _tc_edge(hs, hr, r3, qre, p["gW1"], p["gb1"].reshape(1, -1),
                        p["gW2"], p["gb2"].reshape(1, -1), p["Ws"], p["Wqr"],
                        p["bqr"].reshape(1, -1), p["wa"],
                        p["ba"].reshape(1, -1))
        aggx = _sc_scatter(msgx, obj_sc, zrows)
        hid, sc = _tc_node(aggx[:NNODE], ng3, h0_in, qre, p["Wh"],
                           tp[0], tp[1], tp[2], tp[3], Wf)
        return hid, sc

    zeros_h = jnp.zeros((NNODE, HID), jnp.float32)
    hidden, _ = run_layer(layers[0], zeros_h, zeros_h, sub_sc)
    h0 = hidden
    h0_in = jnp.zeros((NNODE, HID), jnp.float32).at[old_idx].set(h0)
    hidden, _ = run_layer(layers[1], hidden, h0_in, sub_sc)
    h0 = hidden
    for i in range(NXL):
        hidden, sc = run_layer(layers[NL + i], hidden, h0, sub2_sc)
        h0 = hidden

    scores = sc[:, 0]
    scores_all = jnp.zeros((n, NNODE), jnp.float32).at[
        nodes[:, 0], nodes[:, 1]].set(scores)
    return scores_all


# speedup vs baseline: 1.6899x; 1.0418x over previous
import functools

import jax
import jax.numpy as jnp
from jax import lax
from jax.experimental import pallas as pl
from jax.experimental.pallas import tpu as pltpu
from jax.experimental.pallas import tpu_sc as plsc

HID = 128
ATT = 5
NVOC = 475
NNODE = 10000
NQ = 16
NE = 160000
NL = 2
NXL = 4

NC = 2
NS = 16
NW = NC * NS
CH = 128
K = 40
NEP = NW * K * CH
DM = 160
DMH = 80
NSEG = 10016
BE = 1280
BN = 2000

@functools.cache
def _mesh():
    return plsc.VectorSubcoreMesh(core_axis_name="c", subcore_axis_name="s",
                                  num_cores=NC, num_subcores=NS)


_SC_PARAMS = pltpu.CompilerParams(use_tc_tiling_on_sc=False)


def _wid():
    return lax.axis_index("s") * NC + lax.axis_index("c")


def _gather2_body(tab1, tab2, idx1, idx2, out1, out2,
                  idxb, rows, gs0, gs1, ss0, ss1):
    w = _wid()

    def run_table(tab, idx_hbm, out):
        pltpu.sync_copy(idx_hbm.at[w], idxb)
        base = w * (K * CH)
        gsem = (gs0, gs1)
        ssem = (ss0, ss1)

        def gcp(j, slot, sem):
            return pltpu.make_async_copy(tab.at[idxb.at[j]], rows.at[slot],
                                         sem)

        def scp(j, slot, sem):
            return pltpu.make_async_copy(
                rows.at[slot], out.at[pl.ds(base + j * CH, CH)], sem)

        def fire_g(g, jbase):
            gcp(jbase, 2 * (g % 2), gsem[g % 2]).start()
            gcp(jbase + 1, 2 * (g % 2) + 1, gsem[g % 2]).start()

        def drain_g(g, jbase):
            gcp(jbase, 2 * (g % 2), gsem[g % 2]).wait()
            gcp(jbase + 1, 2 * (g % 2) + 1, gsem[g % 2]).wait()

        def fire_s(g, jbase):
            scp(jbase, 2 * (g % 2), ssem[g % 2]).start()
            scp(jbase + 1, 2 * (g % 2) + 1, ssem[g % 2]).start()

        def drain_s(g, jbase):
            scp(jbase, 2 * (g % 2), ssem[g % 2]).wait()
            scp(jbase + 1, 2 * (g % 2) + 1, ssem[g % 2]).wait()

        ngrp = K // 2
        fire_g(0, 0)
        fire_g(1, 2)

        def body(i, _):
            ja = 4 * i
            jb = ja + 2
            drain_g(0, ja)
            fire_s(0, ja)
            drain_g(1, jb)
            fire_s(1, jb)

            @pl.when(i < ngrp // 2 - 1)
            def _():
                drain_s(0, ja)
                fire_g(0, ja + 4)
                drain_s(1, jb)
                fire_g(1, jb + 4)

            @pl.when(i == ngrp // 2 - 1)
            def _():
                drain_s(0, ja)
                drain_s(1, jb)

            return 0

        lax.fori_loop(0, ngrp // 2, body, 0)

    run_table(tab1, idx1, out1)
    run_table(tab2, idx2, out2)


@functools.partial(jax.jit, static_argnames=())
def _sc_gather2(tab1, tab2, idx1, idx2):
    f = pl.kernel(
        _gather2_body,
        out_type=[jax.ShapeDtypeStruct((NEP, HID), jnp.float32),
                  jax.ShapeDtypeStruct((NEP, HID), jnp.float32)],
        mesh=_mesh(),
        scratch_types=[pltpu.VMEM((K, CH), jnp.int32),
                       pltpu.VMEM((4, CH, HID), jnp.float32),
                       pltpu.SemaphoreType.DMA,
                       pltpu.SemaphoreType.DMA,
                       pltpu.SemaphoreType.DMA,
                       pltpu.SemaphoreType.DMA],
        compiler_params=_SC_PARAMS,
    )
    return f(tab1, tab2, idx1, idx2)


def _scatter_body(msgx, obj_idx, zrows, out, idxb, rows, ls0, ls1, as0, as1,
                  shared):
    c = lax.axis_index("c")
    s = lax.axis_index("s")
    seg_per_tile = NSEG // NS
    t0 = s * seg_per_tile
    col0 = c * DMH

    pltpu.sync_copy(zrows, shared.at[pl.ds(t0, seg_per_tile)])
    plsc.subcore_barrier()

    def run_span(w):
        pltpu.sync_copy(obj_idx.at[w], idxb)
        base = w * (K * CH)
        lsem = (ls0, ls1)
        asem = (as0, as1)

        def lcp(j, slot, sem):
            return pltpu.make_async_copy(
                msgx.at[pl.ds(base + j * CH, CH), pl.ds(col0, DMH)],
                rows.at[slot], sem)

        def acp(j, slot, sem):
            return pltpu.make_async_copy(rows.at[slot],
                                         shared.at[idxb.at[j]], sem)

        def fire_l(g, jbase):
            lcp(jbase, 2 * (g % 2), lsem[g % 2]).start()
            lcp(jbase + 1, 2 * (g % 2) + 1, lsem[g % 2]).start()

        def drain_l(g, jbase):
            lcp(jbase, 2 * (g % 2), lsem[g % 2]).wait()
            lcp(jbase + 1, 2 * (g % 2) + 1, lsem[g % 2]).wait()

        def fire_a(g, jbase):
            pltpu.async_copy(rows.at[2 * (g % 2)],
                             shared.at[idxb.at[jbase]], asem[g % 2],
                             add=True)
            pltpu.async_copy(rows.at[2 * (g % 2) + 1],
                             shared.at[idxb.at[jbase + 1]], asem[g % 2],
                             add=True)

        def drain_a(g, jbase):
            acp(jbase, 2 * (g % 2), asem[g % 2]).wait()
            acp(jbase + 1, 2 * (g % 2) + 1, asem[g % 2]).wait()

        ngrp = K // 2
        fire_l(0, 0)
        fire_l(1, 2)

        def body(i, _):
            ja = 4 * i
            jb = ja + 2
            drain_l(0, ja)
            fire_a(0, ja)
            drain_l(1, jb)
            fire_a(1, jb)

            @pl.when(i < ngrp // 2 - 1)
            def _():
                drain_a(0, ja)
                fire_l(0, ja + 4)
                drain_a(1, jb)
                fire_l(1, jb + 4)

            @pl.when(i == ngrp // 2 - 1)
            def _():
                drain_a(0, ja)
                drain_a(1, jb)

            return 0

        lax.fori_loop(0, ngrp // 2, body, 0)

    run_span(2 * s)
    run_span(2 * s + 1)
    plsc.subcore_barrier()
    pltpu.sync_copy(shared.at[pl.ds(t0, seg_per_tile)],
                    out.at[pl.ds(t0, seg_per_tile), pl.ds(col0, DMH)])


@jax.jit
def _sc_scatter(msgx, obj_idx, zrows):
    f = pl.kernel(
        _scatter_body,
        out_type=jax.ShapeDtypeStruct((NSEG, DM), jnp.float32),
        mesh=_mesh(),
        scratch_types=[pltpu.VMEM((K, CH), jnp.int32),
                       pltpu.VMEM((4, CH, DMH), jnp.float32),
                       pltpu.SemaphoreType.DMA,
                       pltpu.SemaphoreType.DMA,
                       pltpu.SemaphoreType.DMA,
                       pltpu.SemaphoreType.DMA,
                       pltpu.VMEM_SHARED((NSEG, DMH), jnp.float32)],
        compiler_params=_SC_PARAMS,
    )
    return f(msgx, obj_idx, zrows)


def _edge_body(hs_ref, hr_ref, r_ref, qre_ref, gW1_ref, gb1_ref, gW2_ref,
               gb2_ref, Ws_ref, Wqr_ref, bqr_ref, wa_ref, ba_ref, out_ref):
    hs = hs_ref[...]
    hr = hr_ref[...]
    r = r_ref[0]
    iot = lax.broadcasted_iota(jnp.int32, (NQ, BE), 0)
    oh = (iot == r).astype(jnp.float32)
    qre = qre_ref[...]
    dn = (((0,), (0,)), ((), ()))
    hqr = lax.dot_general(oh, qre, dn, preferred_element_type=jnp.float32)
    gW1 = gW1_ref[...]
    g = hr @ gW1[:HID] + hqr @ gW1[HID:2 * HID] + hs @ gW1[2 * HID:]
    g = 1.0 / (1.0 + jnp.exp(-(g + gb1_ref[...])))
    u = g[:, :HID]
    rs = g[:, HID:]
    gW2 = gW2_ref[...]
    cand = jnp.tanh(hr @ gW2[:HID] + (rs * hs) @ gW2[HID:] + gb2_ref[...])
    uri = (1.0 - u) * hs + u * cand
    qa = qre @ Wqr_ref[...] + bqr_ref[...]
    s = jnp.maximum(uri @ Ws_ref[...] +
                    lax.dot_general(oh, qa, dn,
                                    preferred_element_type=jnp.float32), 0.0)
    aw = s @ wa_ref[...] + ba_ref[...]
    ue = jnp.exp(aw)
    out_ref[...] = jnp.concatenate(
        [ue * uri, ue, jnp.zeros((BE, DM - HID - 1), jnp.float32)], axis=1)


def _full(shape):
    nd = len(shape)
    return pl.BlockSpec(shape, lambda i, _n=nd: (0,) * _n)


@jax.jit
def _tc_edge(hs, hr, r3, qre, gW1, gb1, gW2, gb2, Ws, Wqr, bqr, wa, ba):
    nblk = NEP // BE
    return pl.pallas_call(
        _edge_body,
        out_shape=jax.ShapeDtypeStruct((NEP, DM), jnp.float32),
        grid=(nblk,),
        in_specs=[
            pl.BlockSpec((BE, HID), lambda i: (i, 0)),
            pl.BlockSpec((BE, HID), lambda i: (i, 0)),
            pl.BlockSpec((1, 1, BE), lambda i: (i, 0, 0)),
            _full((NQ, HID)), _full((3 * HID, 2 * HID)), _full((1, 2 * HID)),
            _full((2 * HID, HID)), _full((1, HID)), _full((HID, ATT)),
            _full((HID, ATT)), _full((1, ATT)), _full((ATT, 1)),
            _full((1, 1)),
        ],
        out_specs=pl.BlockSpec((BE, DM), lambda i: (i, 0)),
    )(hs, hr, r3, qre, gW1, gb1, gW2, gb2, Ws, Wqr, bqr, wa, ba)


def _node_body(a0_ref, ng_ref, h0_ref, qre_ref, Wh_ref, gW1_ref,
               gb1_ref, gW2_ref, gb2_ref, Wf_ref, hid_ref, sc_ref):
    a = a0_ref[...]
    agg = a[:, :HID]
    se = a[:, HID:HID + 1]
    m = jnp.maximum((agg / se) @ Wh_ref[...], 0.0)
    ng = ng_ref[0]
    iot = lax.broadcasted_iota(jnp.int32, (NQ, BN), 0)
    oh = (iot == ng).astype(jnp.float32)
    dn = (((0,), (0,)), ((), ()))
    hqr = lax.dot_general(oh, qre_ref[...], dn,
                          preferred_element_type=jnp.float32)
    h = h0_ref[...]
    gW1 = gW1_ref[...]
    g = m @ gW1[:HID] + hqr @ gW1[HID:2 * HID] + h @ gW1[2 * HID:]
    g = 1.0 / (1.0 + jnp.exp(-(g + gb1_ref[...])))
    u = g[:, :HID]
    rs = g[:, HID:]
    gW2 = gW2_ref[...]
    cand = jnp.tanh(m @ gW2[:HID] + (rs * h) @ gW2[HID:] + gb2_ref[...])
    out = (1.0 - u) * h + u * cand
    hid_ref[...] = out
    sc_ref[...] = out @ Wf_ref[...]


@jax.jit
def _tc_node(a0, ng3, h0, qre, Wh, gW1, gb1, gW2, gb2, Wf):
    nblk = NNODE // BN
    return pl.pallas_call(
        _node_body,
        out_shape=[jax.ShapeDtypeStruct((NNODE, HID), jnp.float32),
                   jax.ShapeDtypeStruct((NNODE, 1), jnp.float32)],
        grid=(nblk,),
        in_specs=[
            pl.BlockSpec((BN, DM), lambda i: (i, 0)),
            pl.BlockSpec((1, 1, BN), lambda i: (i, 0, 0)),
            pl.BlockSpec((BN, HID), lambda i: (i, 0)),
            _full((NQ, HID)), _full((HID, HID)), _full((3 * HID, 2 * HID)),
            _full((1, 2 * HID)), _full((2 * HID, HID)), _full((1, HID)),
            _full((HID, 1)),
        ],
        out_specs=[pl.BlockSpec((BN, HID), lambda i: (i, 0)),
                   pl.BlockSpec((BN, 1), lambda i: (i, 0))],
    )(a0, ng3, h0, qre, Wh, gW1, gb1, gW2, gb2, Wf)


def kernel(subs, rels, edges, nodes, old_idx, params):
    n = subs.shape[0]
    idt = edges.dtype
    r_idx = edges[:, 0]
    rel = edges[:, 2]
    sub = edges[:, 4]
    obj = edges[:, 5]
    sub2 = old_idx[sub]

    node_group = jnp.zeros((NNODE,), dtype=idt).at[obj].set(r_idx)

    pad = NEP - NE
    def p32(x, fill):
        return jnp.concatenate(
            [x, jnp.full((pad,), fill, x.dtype)]).reshape(NW, K, CH)

    sub_sc = p32(sub, 0)
    sub2_sc = p32(sub2, 0)
    rel_sc = p32(rel, 0)
    obj_sc = p32(obj, NNODE)
    r3 = jnp.concatenate([r_idx, jnp.zeros((pad,), idt)]).reshape(
        NEP // BE, 1, BE)
    ng3 = node_group.reshape(NNODE // BN, 1, BN)
    zrows = jnp.zeros((NSEG // NS, DMH), jnp.float32)

    layers = params["layers"]
    top = params["top"]
    tp = [top["gW1"], top["gb1"].reshape(1, -1),
          top["gW2"], top["gb2"].reshape(1, -1)]
    Wf = params["Wfinal"]

    def run_layer(p, hidden, h0_in, sub_idx):
        qre = p["rela"][rels]
        hs, hr = _sc_gather2(hidden, p["rela"], sub_idx, rel_sc)
        msgx = _tc_edge(hs, hr, r3, qre, p["gW1"], p["gb1"].reshape(1, -1),
                        p["gW2"], p["gb2"].reshape(1, -1), p["Ws"], p["Wqr"],
                        p["bqr"].reshape(1, -1), p["wa"],
                        p["ba"].reshape(1, -1))
        aggx = _sc_scatter(msgx, obj_sc, zrows)
        hid, sc = _tc_node(aggx[:NNODE], ng3, h0_in, qre, p["Wh"],
                           tp[0], tp[1], tp[2], tp[3], Wf)
        return hid, sc

    zeros_h = jnp.zeros((NNODE, HID), jnp.float32)
    hidden, _ = run_layer(layers[0], zeros_h, zeros_h, sub_sc)
    h0 = hidden
    h0_in = jnp.zeros((NNODE, HID), jnp.float32).at[old_idx].set(h0)
    hidden, _ = run_layer(layers[1], hidden, h0_in, sub_sc)
    h0 = hidden
    for i in range(NXL):
        hidden, sc = run_layer(layers[NL + i], hidden, h0, sub2_sc)
        h0 = hidden

    scores = sc[:, 0]
    scores_all = jnp.zeros((n, NNODE), jnp.float32).at[
        nodes[:, 0], nodes[:, 1]].set(scores)
    return scores_all


# speedup vs baseline: 1.9562x; 1.1576x over previous
import functools

import jax
import jax.numpy as jnp
from jax import lax
from jax.experimental import pallas as pl
from jax.experimental.pallas import tpu as pltpu
from jax.experimental.pallas import tpu_sc as plsc

HID = 128
ATT = 5
NVOC = 475
NNODE = 10000
NQ = 16
NE = 160000
NL = 2
NXL = 4

NC = 2
NS = 16
NW = NC * NS
CH = 128
K = 40
NEP = NW * K * CH
DM = 160
DMH = 80
NSEG = 10016
BE = 1280
BN = 2000
NKEY = NVOC * NQ
KEYP = 7680


@functools.cache
def _mesh():
    return plsc.VectorSubcoreMesh(core_axis_name="c", subcore_axis_name="s",
                                  num_cores=NC, num_subcores=NS)


_SC_PARAMS = pltpu.CompilerParams(use_tc_tiling_on_sc=False)


def _wid():
    return lax.axis_index("s") * NC + lax.axis_index("c")


def _gather_body(tab, idx_hbm, out, idxb, rows, gs0, gs1, ss0, ss1):
    w = _wid()
    pltpu.sync_copy(idx_hbm.at[w], idxb)
    base = w * (K * CH)
    gsem = (gs0, gs1)
    ssem = (ss0, ss1)

    def gcp(j, slot, sem):
        return pltpu.make_async_copy(tab.at[idxb.at[j]], rows.at[slot], sem)

    def scp(j, slot, sem):
        return pltpu.make_async_copy(rows.at[slot],
                                     out.at[pl.ds(base + j * CH, CH)], sem)

    def fire_g(g, jbase):
        gcp(jbase, 2 * g, gsem[g]).start()
        gcp(jbase + 1, 2 * g + 1, gsem[g]).start()

    def drain_g(g, jbase):
        gcp(jbase, 2 * g, gsem[g]).wait()
        gcp(jbase + 1, 2 * g + 1, gsem[g]).wait()

    def fire_s(g, jbase):
        scp(jbase, 2 * g, ssem[g]).start()
        scp(jbase + 1, 2 * g + 1, ssem[g]).start()

    def drain_s(g, jbase):
        scp(jbase, 2 * g, ssem[g]).wait()
        scp(jbase + 1, 2 * g + 1, ssem[g]).wait()

    nit = K // 4
    fire_g(0, 0)
    fire_g(1, 2)

    def body(i, _):
        ja = 4 * i
        jb = ja + 2
        drain_g(0, ja)
        fire_s(0, ja)
        drain_g(1, jb)
        fire_s(1, jb)

        @pl.when(i < nit - 1)
        def _():
            drain_s(0, ja)
            fire_g(0, ja + 4)
            drain_s(1, jb)
            fire_g(1, jb + 4)

        @pl.when(i == nit - 1)
        def _():
            drain_s(0, ja)
            drain_s(1, jb)

        return 0

    lax.fori_loop(0, nit, body, 0)


@jax.jit
def _sc_gather(tab, idx):
    dt = tab.dtype
    f = pl.kernel(
        _gather_body,
        out_type=jax.ShapeDtypeStruct((NEP, HID), dt),
        mesh=_mesh(),
        scratch_types=[pltpu.VMEM((K, CH), jnp.int32),
                       pltpu.VMEM((4, CH, HID), dt),
                       pltpu.SemaphoreType.DMA,
                       pltpu.SemaphoreType.DMA,
                       pltpu.SemaphoreType.DMA,
                       pltpu.SemaphoreType.DMA],
        compiler_params=_SC_PARAMS,
    )
    return f(tab, idx)


def _scatter_body(msgx, obj_idx, zrows, out, idxb, rows, ls0, ls1, as0, as1,
                  shared):
    c = lax.axis_index("c")
    s = lax.axis_index("s")
    seg_per_tile = NSEG // NS
    t0 = s * seg_per_tile
    col0 = c * DMH

    pltpu.sync_copy(zrows, shared.at[pl.ds(t0, seg_per_tile)])
    plsc.subcore_barrier()

    def run_span(w):
        pltpu.sync_copy(obj_idx.at[w], idxb)
        base = w * (K * CH)
        lsem = (ls0, ls1)
        asem = (as0, as1)

        def lcp(j, slot, sem):
            return pltpu.make_async_copy(
                msgx.at[pl.ds(base + j * CH, CH), pl.ds(col0, DMH)],
                rows.at[slot], sem)

        def acp(j, slot, sem):
            return pltpu.make_async_copy(rows.at[slot],
                                         shared.at[idxb.at[j]], sem)

        def fire_l(g, jbase):
            lcp(jbase, 2 * g, lsem[g]).start()
            lcp(jbase + 1, 2 * g + 1, lsem[g]).start()

        def drain_l(g, jbase):
            lcp(jbase, 2 * g, lsem[g]).wait()
            lcp(jbase + 1, 2 * g + 1, lsem[g]).wait()

        def fire_a(g, jbase):
            pltpu.async_copy(rows.at[2 * g], shared.at[idxb.at[jbase]],
                             asem[g], add=True)
            pltpu.async_copy(rows.at[2 * g + 1],
                             shared.at[idxb.at[jbase + 1]], asem[g],
                             add=True)

        def drain_a(g, jbase):
            acp(jbase, 2 * g, asem[g]).wait()
            acp(jbase + 1, 2 * g + 1, asem[g]).wait()

        nit = K // 4
        fire_l(0, 0)
        fire_l(1, 2)

        def body(i, _):
            ja = 4 * i
            jb = ja + 2
            drain_l(0, ja)
            fire_a(0, ja)
            drain_l(1, jb)
            fire_a(1, jb)

            @pl.when(i < nit - 1)
            def _():
                drain_a(0, ja)
                fire_l(0, ja + 4)
                drain_a(1, jb)
                fire_l(1, jb + 4)

            @pl.when(i == nit - 1)
            def _():
                drain_a(0, ja)
                drain_a(1, jb)

            return 0

        lax.fori_loop(0, nit, body, 0)

    run_span(2 * s)
    run_span(2 * s + 1)
    plsc.subcore_barrier()
    pltpu.sync_copy(shared.at[pl.ds(t0, seg_per_tile)],
                    out.at[pl.ds(t0, seg_per_tile), pl.ds(col0, DMH)])


@jax.jit
def _sc_scatter(msgx, obj_idx, zrows):
    f = pl.kernel(
        _scatter_body,
        out_type=jax.ShapeDtypeStruct((NSEG, DM), jnp.float32),
        mesh=_mesh(),
        scratch_types=[pltpu.VMEM((K, CH), jnp.int32),
                       pltpu.VMEM((4, CH, DMH), jnp.float32),
                       pltpu.SemaphoreType.DMA,
                       pltpu.SemaphoreType.DMA,
                       pltpu.SemaphoreType.DMA,
                       pltpu.SemaphoreType.DMA,
                       pltpu.VMEM_SHARED((NSEG, DMH), jnp.float32)],
        compiler_params=_SC_PARAMS,
    )
    return f(msgx, obj_idx, zrows)


def _l0_body(m0s, key_idx, obj_idx, zrows, out, keyb, objb, rows,
             ls0, ls1, as0, as1, shared):
    c = lax.axis_index("c")
    s = lax.axis_index("s")
    seg_per_tile = NSEG // NS
    t0 = s * seg_per_tile

    pltpu.sync_copy(zrows, shared.at[pl.ds(t0, seg_per_tile)])
    plsc.subcore_barrier()

    m0c = m0s.at[c]

    def run_span(w):
        pltpu.sync_copy(key_idx.at[w], keyb)
        pltpu.sync_copy(obj_idx.at[w], objb)
        lsem = (ls0, ls1)
        asem = (as0, as1)

        def lcp(j, slot, sem):
            return pltpu.make_async_copy(m0c.at[keyb.at[j]], rows.at[slot],
                                         sem)

        def acp(j, slot, sem):
            return pltpu.make_async_copy(rows.at[slot],
                                         shared.at[objb.at[j]], sem)

        def fire_l(g, jbase):
            lcp(jbase, 2 * g, lsem[g]).start()
            lcp(jbase + 1, 2 * g + 1, lsem[g]).start()

        def drain_l(g, jbase):
            lcp(jbase, 2 * g, lsem[g]).wait()
            lcp(jbase + 1, 2 * g + 1, lsem[g]).wait()

        def fire_a(g, jbase):
            pltpu.async_copy(rows.at[2 * g], shared.at[objb.at[jbase]],
                             asem[g], add=True)
            pltpu.async_copy(rows.at[2 * g + 1],
                             shared.at[objb.at[jbase + 1]], asem[g],
                             add=True)

        def drain_a(g, jbase):
            acp(jbase, 2 * g, asem[g]).wait()
            acp(jbase + 1, 2 * g + 1, asem[g]).wait()

        nit = K // 4
        fire_l(0, 0)
        fire_l(1, 2)

        def body(i, _):
            ja = 4 * i
            jb = ja + 2
            drain_l(0, ja)
            fire_a(0, ja)
            drain_l(1, jb)
            fire_a(1, jb)

            @pl.when(i < nit - 1)
            def _():
                drain_a(0, ja)
                fire_l(0, ja + 4)
                drain_a(1, jb)
                fire_l(1, jb + 4)

            @pl.when(i == nit - 1)
            def _():
                drain_a(0, ja)
                drain_a(1, jb)

            return 0

        lax.fori_loop(0, nit, body, 0)

    run_span(2 * s)
    run_span(2 * s + 1)
    plsc.subcore_barrier()
    pltpu.sync_copy(shared.at[pl.ds(t0, seg_per_tile)],
                    out.at[pl.ds(t0, seg_per_tile), pl.ds(c * DMH, DMH)])


@jax.jit
def _sc_l0(m0s, key_idx, obj_idx, zrows):
    f = pl.kernel(
        _l0_body,
        out_type=jax.ShapeDtypeStruct((NSEG, DM), jnp.float32),
        mesh=_mesh(),
        scratch_types=[pltpu.VMEM((K, CH), jnp.int32),
                       pltpu.VMEM((K, CH), jnp.int32),
                       pltpu.VMEM((4, CH, DMH), jnp.float32),
                       pltpu.SemaphoreType.DMA,
                       pltpu.SemaphoreType.DMA,
                       pltpu.SemaphoreType.DMA,
                       pltpu.SemaphoreType.DMA,
                       pltpu.VMEM_SHARED((NSEG, DMH), jnp.float32)],
        compiler_params=_SC_PARAMS,
    )
    return f(m0s, key_idx, obj_idx, zrows)


def _edge_body(hs_ref, hr_ref, r_ref, qre_ref, gW1_ref, gb1_ref, gW2_ref,
               gb2_ref, Ws_ref, Wqr_ref, bqr_ref, wa_ref, ba_ref, out_ref):
    hs = hs_ref[...].astype(jnp.float32)
    hr = hr_ref[...].astype(jnp.float32)
    nb = hs.shape[0]
    r = r_ref[0]
    iot = lax.broadcasted_iota(jnp.int32, (NQ, nb), 0)
    oh = (iot == r).astype(jnp.float32)
    qre = qre_ref[...]
    dn = (((0,), (0,)), ((), ()))
    hqr = lax.dot_general(oh, qre, dn, preferred_element_type=jnp.float32)
    gW1 = gW1_ref[...]
    g = hr @ gW1[:HID] + hqr @ gW1[HID:2 * HID] + hs @ gW1[2 * HID:]
    g = 1.0 / (1.0 + jnp.exp(-(g + gb1_ref[...])))
    u = g[:, :HID]
    rs = g[:, HID:]
    gW2 = gW2_ref[...]
    cand = jnp.tanh(hr @ gW2[:HID] + (rs * hs) @ gW2[HID:] + gb2_ref[...])
    uri = (1.0 - u) * hs + u * cand
    qa = qre @ Wqr_ref[...] + bqr_ref[...]
    sc = jnp.maximum(uri @ Ws_ref[...] +
                     lax.dot_general(oh, qa, dn,
                                     preferred_element_type=jnp.float32), 0.0)
    aw = sc @ wa_ref[...] + ba_ref[...]
    ue = jnp.exp(aw)
    out_ref[...] = jnp.concatenate(
        [ue * uri, ue, jnp.zeros((nb, DM - HID - 1), jnp.float32)], axis=1)


def _full(shape):
    nd = len(shape)
    return pl.BlockSpec(shape, lambda i, _n=nd: (0,) * _n)


@jax.jit
def _tc_edge(hs, hr, r3, qre, gW1, gb1, gW2, gb2, Ws, Wqr, bqr, wa, ba):
    ne = hs.shape[0]
    nblk = ne // BE
    return pl.pallas_call(
        _edge_body,
        out_shape=jax.ShapeDtypeStruct((ne, DM), jnp.float32),
        grid=(nblk,),
        in_specs=[
            pl.BlockSpec((BE, HID), lambda i: (i, 0)),
            pl.BlockSpec((BE, HID), lambda i: (i, 0)),
            pl.BlockSpec((1, 1, BE), lambda i: (i, 0, 0)),
            _full((NQ, HID)), _full((3 * HID, 2 * HID)), _full((1, 2 * HID)),
            _full((2 * HID, HID)), _full((1, HID)), _full((HID, ATT)),
            _full((HID, ATT)), _full((1, ATT)), _full((ATT, 1)),
            _full((1, 1)),
        ],
        out_specs=pl.BlockSpec((BE, DM), lambda i: (i, 0)),
    )(hs, hr, r3, qre, gW1, gb1, gW2, gb2, Ws, Wqr, bqr, wa, ba)


def _node_body(a0_ref, ng_ref, h0_ref, qre_ref, Wh_ref, gW1_ref,
               gb1_ref, gW2_ref, gb2_ref, Wf_ref, hid_ref, hbf_ref, sc_ref):
    a = a0_ref[...]
    agg = a[:, :HID]
    se = a[:, HID:HID + 1]
    m = jnp.maximum((agg / se) @ Wh_ref[...], 0.0)
    ng = ng_ref[0]
    iot = lax.broadcasted_iota(jnp.int32, (NQ, BN), 0)
    oh = (iot == ng).astype(jnp.float32)
    dn = (((0,), (0,)), ((), ()))
    hqr = lax.dot_general(oh, qre_ref[...], dn,
                          preferred_element_type=jnp.float32)
    h = h0_ref[...]
    gW1 = gW1_ref[...]
    g = m @ gW1[:HID] + hqr @ gW1[HID:2 * HID] + h @ gW1[2 * HID:]
    g = 1.0 / (1.0 + jnp.exp(-(g + gb1_ref[...])))
    u = g[:, :HID]
    rs = g[:, HID:]
    gW2 = gW2_ref[...]
    cand = jnp.tanh(m @ gW2[:HID] + (rs * h) @ gW2[HID:] + gb2_ref[...])
    out = (1.0 - u) * h + u * cand
    hid_ref[...] = out
    hbf_ref[...] = out.astype(jnp.bfloat16)
    sc_ref[...] = out @ Wf_ref[...]


@jax.jit
def _tc_node(a0, ng3, h0, qre, Wh, gW1, gb1, gW2, gb2, Wf):
    nblk = NNODE // BN
    return pl.pallas_call(
        _node_body,
        out_shape=[jax.ShapeDtypeStruct((NNODE, HID), jnp.float32),
                   jax.ShapeDtypeStruct((NNODE, HID), jnp.bfloat16),
                   jax.ShapeDtypeStruct((NNODE, 1), jnp.float32)],
        grid=(nblk,),
        in_specs=[
            pl.BlockSpec((BN, DM), lambda i: (i, 0)),
            pl.BlockSpec((1, 1, BN), lambda i: (i, 0, 0)),
            pl.BlockSpec((BN, HID), lambda i: (i, 0)),
            _full((NQ, HID)), _full((HID, HID)), _full((3 * HID, 2 * HID)),
            _full((1, 2 * HID)), _full((2 * HID, HID)), _full((1, HID)),
            _full((HID, 1)),
        ],
        out_specs=[pl.BlockSpec((BN, HID), lambda i: (i, 0)),
                   pl.BlockSpec((BN, HID), lambda i: (i, 0)),
                   pl.BlockSpec((BN, 1), lambda i: (i, 0))],
    )(a0, ng3, h0, qre, Wh, gW1, gb1, gW2, gb2, Wf)


def kernel(subs, rels, edges, nodes, old_idx, params):
    n = subs.shape[0]
    idt = edges.dtype
    r_idx = edges[:, 0]
    rel = edges[:, 2]
    sub = edges[:, 4]
    obj = edges[:, 5]
    sub2 = old_idx[sub]
    key = rel * NQ + r_idx

    node_group = jnp.zeros((NNODE,), dtype=idt).at[obj].set(r_idx)

    pad = NEP - NE

    def p32(x, fill):
        return jnp.concatenate(
            [x, jnp.full((pad,), fill, x.dtype)]).reshape(NW, K, CH)

    sub_sc = p32(sub, 0)
    sub2_sc = p32(sub2, 0)
    rel_sc = p32(rel, 0)
    obj_sc = p32(obj, NNODE)
    key_sc = p32(key, 0)
    r3 = jnp.concatenate([r_idx, jnp.zeros((pad,), idt)]).reshape(
        NEP // BE, 1, BE)
    ng3 = node_group.reshape(NNODE // BN, 1, BN)
    zrows = jnp.zeros((NSEG // NS, DMH), jnp.float32)

    layers = params["layers"]
    top = params["top"]
    tp = [top["gW1"], top["gb1"].reshape(1, -1),
          top["gW2"], top["gb2"].reshape(1, -1)]
    Wf = params["Wfinal"]

    def edge_call(p, hs, hr, r3v, qre):
        return _tc_edge(hs, hr, r3v, qre, p["gW1"], p["gb1"].reshape(1, -1),
                        p["gW2"], p["gb2"].reshape(1, -1), p["Ws"], p["Wqr"],
                        p["bqr"].reshape(1, -1), p["wa"],
                        p["ba"].reshape(1, -1))

    def node_call(p, aggx, h0_in, qre):
        return _tc_node(aggx[:NNODE], ng3, h0_in, qre, p["Wh"],
                        tp[0], tp[1], tp[2], tp[3], Wf)

    qres = [p["rela"][rels] for p in layers]

    hr_l = [None] + [
        _sc_gather(layers[li]["rela"].astype(jnp.bfloat16), rel_sc)
        for li in range(1, NL + NXL)]

    p0 = layers[0]
    hs0 = jnp.zeros((KEYP, HID), jnp.bfloat16)
    hr0 = jnp.concatenate(
        [jnp.repeat(p0["rela"], NQ, axis=0),
         jnp.zeros((KEYP - NKEY, HID), jnp.float32)]).astype(jnp.bfloat16)
    r0 = jnp.tile(jnp.arange(NQ, dtype=idt), KEYP // NQ).reshape(
        KEYP // BE, 1, BE)
    m0 = edge_call(p0, hs0, hr0, r0, qres[0])
    m0s = jnp.stack([m0[:, :DMH], m0[:, DMH:]])
    aggx = _sc_l0(m0s, key_sc, obj_sc, zrows)
    zeros_h = jnp.zeros((NNODE, HID), jnp.float32)
    hidden, hidden_bf, _ = node_call(p0, aggx, zeros_h, qres[0])
    h0 = hidden
    h0_in = jnp.zeros((NNODE, HID), jnp.float32).at[old_idx].set(h0)

    for li in range(1, NL + NXL):
        p = layers[li]
        sub_idx = sub_sc if li < NL else sub2_sc
        hs = _sc_gather(hidden_bf, sub_idx)
        msgx = edge_call(p, hs, hr_l[li], r3, qres[li])
        aggx = _sc_scatter(msgx, obj_sc, zrows)
        hidden, hidden_bf, sc = node_call(p, aggx, h0_in, qres[li])
        h0_in = hidden

    scores = sc[:, 0]
    scores_all = jnp.zeros((n, NNODE), jnp.float32).at[
        nodes[:, 0], nodes[:, 1]].set(scores)
    return scores_all


# speedup vs baseline: 2.1169x; 1.0822x over previous
import functools

import jax
import jax.numpy as jnp
from jax import lax
from jax.experimental import pallas as pl
from jax.experimental.pallas import tpu as pltpu
from jax.experimental.pallas import tpu_sc as plsc

HID = 128
ATT = 5
NVOC = 475
NNODE = 10000
NQ = 16
NE = 160000
NL = 2
NXL = 4

NC = 2
NS = 16
NW = NC * NS
CH = 128
K = 40
NEP = NW * K * CH
DM = 160
DMH = 80
NSEG = 10016
BE = 1280
BN = 2000
NKEY = NVOC * NQ
KEYP = 7680


@functools.cache
def _mesh():
    return plsc.VectorSubcoreMesh(core_axis_name="c", subcore_axis_name="s",
                                  num_cores=NC, num_subcores=NS)


_SC_PARAMS = pltpu.CompilerParams(use_tc_tiling_on_sc=False)


def _wid():
    return lax.axis_index("s") * NC + lax.axis_index("c")


def _gather_body(tab, idx_hbm, out, idxb, rows, gs0, gs1, ss0, ss1):
    w = _wid()
    pltpu.sync_copy(idx_hbm.at[w], idxb)
    base = w * (K * CH)
    gsem = (gs0, gs1)
    ssem = (ss0, ss1)

    def gcp(j, slot, sem):
        return pltpu.make_async_copy(tab.at[idxb.at[j]], rows.at[slot], sem)

    def scp(j, slot, sem):
        return pltpu.make_async_copy(rows.at[slot],
                                     out.at[pl.ds(base + j * CH, CH)], sem)

    def fire_g(g, jbase):
        gcp(jbase, 2 * g, gsem[g]).start()
        gcp(jbase + 1, 2 * g + 1, gsem[g]).start()

    def drain_g(g, jbase):
        gcp(jbase, 2 * g, gsem[g]).wait()
        gcp(jbase + 1, 2 * g + 1, gsem[g]).wait()

    def fire_s(g, jbase):
        scp(jbase, 2 * g, ssem[g]).start()
        scp(jbase + 1, 2 * g + 1, ssem[g]).start()

    def drain_s(g, jbase):
        scp(jbase, 2 * g, ssem[g]).wait()
        scp(jbase + 1, 2 * g + 1, ssem[g]).wait()

    nit = K // 4
    fire_g(0, 0)
    fire_g(1, 2)

    def body(i, _):
        ja = 4 * i
        jb = ja + 2
        drain_g(0, ja)
        fire_s(0, ja)
        drain_g(1, jb)
        fire_s(1, jb)

        @pl.when(i < nit - 1)
        def _():
            drain_s(0, ja)
            fire_g(0, ja + 4)
            drain_s(1, jb)
            fire_g(1, jb + 4)

        @pl.when(i == nit - 1)
        def _():
            drain_s(0, ja)
            drain_s(1, jb)

        return 0

    lax.fori_loop(0, nit, body, 0)


@jax.jit
def _sc_gather(tab, idx):
    dt = tab.dtype
    f = pl.kernel(
        _gather_body,
        out_type=jax.ShapeDtypeStruct((NEP, HID), dt),
        mesh=_mesh(),
        scratch_types=[pltpu.VMEM((K, CH), jnp.int32),
                       pltpu.VMEM((4, CH, HID), dt),
                       pltpu.SemaphoreType.DMA,
                       pltpu.SemaphoreType.DMA,
                       pltpu.SemaphoreType.DMA,
                       pltpu.SemaphoreType.DMA],
        compiler_params=_SC_PARAMS,
    )
    return f(tab, idx)


def _gather_multi_body(*args):
    ntab = (len(args) - 6) // 2
    tabs = args[:ntab]
    idx_hbm = args[ntab]
    outs = args[ntab + 1:2 * ntab + 1]
    idxb, rows, gs0, gs1, ss0, ss1 = args[2 * ntab + 1:]
    w = _wid()
    pltpu.sync_copy(idx_hbm.at[w], idxb)
    base = w * (K * CH)
    gsem = (gs0, gs1)
    ssem = (ss0, ss1)

    for tab, out in zip(tabs, outs):
        def gcp(j, slot, sem):
            return pltpu.make_async_copy(tab.at[idxb.at[j]], rows.at[slot],
                                         sem)

        def scp(j, slot, sem):
            return pltpu.make_async_copy(
                rows.at[slot], out.at[pl.ds(base + j * CH, CH)], sem)

        def fire_g(g, jbase):
            gcp(jbase, 2 * g, gsem[g]).start()
            gcp(jbase + 1, 2 * g + 1, gsem[g]).start()

        def drain_g(g, jbase):
            gcp(jbase, 2 * g, gsem[g]).wait()
            gcp(jbase + 1, 2 * g + 1, gsem[g]).wait()

        def fire_s(g, jbase):
            scp(jbase, 2 * g, ssem[g]).start()
            scp(jbase + 1, 2 * g + 1, ssem[g]).start()

        def drain_s(g, jbase):
            scp(jbase, 2 * g, ssem[g]).wait()
            scp(jbase + 1, 2 * g + 1, ssem[g]).wait()

        nit = K // 4
        fire_g(0, 0)
        fire_g(1, 2)

        def body(i, _):
            ja = 4 * i
            jb = ja + 2
            drain_g(0, ja)
            fire_s(0, ja)
            drain_g(1, jb)
            fire_s(1, jb)

            @pl.when(i < nit - 1)
            def _():
                drain_s(0, ja)
                fire_g(0, ja + 4)
                drain_s(1, jb)
                fire_g(1, jb + 4)

            @pl.when(i == nit - 1)
            def _():
                drain_s(0, ja)
                drain_s(1, jb)

            return 0

        lax.fori_loop(0, nit, body, 0)


@jax.jit
def _sc_gather_multi(tabs, idx):
    dt = tabs[0].dtype
    f = pl.kernel(
        _gather_multi_body,
        out_type=[jax.ShapeDtypeStruct((NEP, HID), dt) for _ in tabs],
        mesh=_mesh(),
        scratch_types=[pltpu.VMEM((K, CH), jnp.int32),
                       pltpu.VMEM((4, CH, HID), dt),
                       pltpu.SemaphoreType.DMA,
                       pltpu.SemaphoreType.DMA,
                       pltpu.SemaphoreType.DMA,
                       pltpu.SemaphoreType.DMA],
        compiler_params=_SC_PARAMS,
    )
    return f(*tabs, idx)


def _scatter_body(msgx, obj_idx, zrows, out, idxb, rows, ls0, ls1, as0, as1,
                  shared):
    c = lax.axis_index("c")
    s = lax.axis_index("s")
    seg_per_tile = NSEG // NS
    t0 = s * seg_per_tile
    col0 = c * DMH

    pltpu.sync_copy(zrows, shared.at[pl.ds(t0, seg_per_tile)])
    plsc.subcore_barrier()

    def run_span(w):
        pltpu.sync_copy(obj_idx.at[w], idxb)
        base = w * (K * CH)
        lsem = (ls0, ls1)
        asem = (as0, as1)

        def lcp(j, slot, sem):
            return pltpu.make_async_copy(
                msgx.at[pl.ds(base + j * CH, CH), pl.ds(col0, DMH)],
                rows.at[slot], sem)

        def acp(j, slot, sem):
            return pltpu.make_async_copy(rows.at[slot],
                                         shared.at[idxb.at[j]], sem)

        def fire_l(g, jbase):
            lcp(jbase, 2 * g, lsem[g]).start()
            lcp(jbase + 1, 2 * g + 1, lsem[g]).start()

        def drain_l(g, jbase):
            lcp(jbase, 2 * g, lsem[g]).wait()
            lcp(jbase + 1, 2 * g + 1, lsem[g]).wait()

        def fire_a(g, jbase):
            pltpu.async_copy(rows.at[2 * g], shared.at[idxb.at[jbase]],
                             asem[g], add=True)
            pltpu.async_copy(rows.at[2 * g + 1],
                             shared.at[idxb.at[jbase + 1]], asem[g],
                             add=True)

        def drain_a(g, jbase):
            acp(jbase, 2 * g, asem[g]).wait()
            acp(jbase + 1, 2 * g + 1, asem[g]).wait()

        nit = K // 4
        fire_l(0, 0)
        fire_l(1, 2)

        def body(i, _):
            ja = 4 * i
            jb = ja + 2
            drain_l(0, ja)
            fire_a(0, ja)
            drain_l(1, jb)
            fire_a(1, jb)

            @pl.when(i < nit - 1)
            def _():
                drain_a(0, ja)
                fire_l(0, ja + 4)
                drain_a(1, jb)
                fire_l(1, jb + 4)

            @pl.when(i == nit - 1)
            def _():
                drain_a(0, ja)
                drain_a(1, jb)

            return 0

        lax.fori_loop(0, nit, body, 0)

    run_span(2 * s)
    run_span(2 * s + 1)
    plsc.subcore_barrier()
    pltpu.sync_copy(shared.at[pl.ds(t0, seg_per_tile)],
                    out.at[pl.ds(t0, seg_per_tile), pl.ds(col0, DMH)])


@jax.jit
def _sc_scatter(msgx, obj_idx, zrows):
    f = pl.kernel(
        _scatter_body,
        out_type=jax.ShapeDtypeStruct((NSEG, DM), jnp.float32),
        mesh=_mesh(),
        scratch_types=[pltpu.VMEM((K, CH), jnp.int32),
                       pltpu.VMEM((4, CH, DMH), jnp.float32),
                       pltpu.SemaphoreType.DMA,
                       pltpu.SemaphoreType.DMA,
                       pltpu.SemaphoreType.DMA,
                       pltpu.SemaphoreType.DMA,
                       pltpu.VMEM_SHARED((NSEG, DMH), jnp.float32)],
        compiler_params=_SC_PARAMS,
    )
    return f(msgx, obj_idx, zrows)


def _l0_body(m0s, key_idx, obj_idx, zrows, out, keyb, objb, rows,
             ls0, ls1, as0, as1, shared):
    c = lax.axis_index("c")
    s = lax.axis_index("s")
    seg_per_tile = NSEG // NS
    t0 = s * seg_per_tile

    pltpu.sync_copy(zrows, shared.at[pl.ds(t0, seg_per_tile)])
    plsc.subcore_barrier()

    m0c = m0s.at[c]

    def run_span(w):
        pltpu.sync_copy(key_idx.at[w], keyb)
        pltpu.sync_copy(obj_idx.at[w], objb)
        lsem = (ls0, ls1)
        asem = (as0, as1)

        def lcp(j, slot, sem):
            return pltpu.make_async_copy(m0c.at[keyb.at[j]], rows.at[slot],
                                         sem)

        def acp(j, slot, sem):
            return pltpu.make_async_copy(rows.at[slot],
                                         shared.at[objb.at[j]], sem)

        def fire_l(g, jbase):
            lcp(jbase, 2 * g, lsem[g]).start()
            lcp(jbase + 1, 2 * g + 1, lsem[g]).start()

        def drain_l(g, jbase):
            lcp(jbase, 2 * g, lsem[g]).wait()
            lcp(jbase + 1, 2 * g + 1, lsem[g]).wait()

        def fire_a(g, jbase):
            pltpu.async_copy(rows.at[2 * g], shared.at[objb.at[jbase]],
                             asem[g], add=True)
            pltpu.async_copy(rows.at[2 * g + 1],
                             shared.at[objb.at[jbase + 1]], asem[g],
                             add=True)

        def drain_a(g, jbase):
            acp(jbase, 2 * g, asem[g]).wait()
            acp(jbase + 1, 2 * g + 1, asem[g]).wait()

        nit = K // 4
        fire_l(0, 0)
        fire_l(1, 2)

        def body(i, _):
            ja = 4 * i
            jb = ja + 2
            drain_l(0, ja)
            fire_a(0, ja)
            drain_l(1, jb)
            fire_a(1, jb)

            @pl.when(i < nit - 1)
            def _():
                drain_a(0, ja)
                fire_l(0, ja + 4)
                drain_a(1, jb)
                fire_l(1, jb + 4)

            @pl.when(i == nit - 1)
            def _():
                drain_a(0, ja)
                drain_a(1, jb)

            return 0

        lax.fori_loop(0, nit, body, 0)

    run_span(2 * s)
    run_span(2 * s + 1)
    plsc.subcore_barrier()
    pltpu.sync_copy(shared.at[pl.ds(t0, seg_per_tile)],
                    out.at[pl.ds(t0, seg_per_tile), pl.ds(c * DMH, DMH)])


@jax.jit
def _sc_l0(m0s, key_idx, obj_idx, zrows):
    f = pl.kernel(
        _l0_body,
        out_type=jax.ShapeDtypeStruct((NSEG, DM), jnp.float32),
        mesh=_mesh(),
        scratch_types=[pltpu.VMEM((K, CH), jnp.int32),
                       pltpu.VMEM((K, CH), jnp.int32),
                       pltpu.VMEM((4, CH, DMH), jnp.float32),
                       pltpu.SemaphoreType.DMA,
                       pltpu.SemaphoreType.DMA,
                       pltpu.SemaphoreType.DMA,
                       pltpu.SemaphoreType.DMA,
                       pltpu.VMEM_SHARED((NSEG, DMH), jnp.float32)],
        compiler_params=_SC_PARAMS,
    )
    return f(m0s, key_idx, obj_idx, zrows)


def _edge_body(hs_ref, hr_ref, r_ref, qre_ref, g1b_ref, g1m_ref, gb1_ref,
               g2b_ref, gb2_ref, Ws_ref, Wqr_ref, bqr_ref, wa_ref, ba_ref,
               out_ref):
    hsb = hs_ref[...]
    hrb = hr_ref[...]
    hs = hsb.astype(jnp.float32)
    nb = hs.shape[0]
    r = r_ref[0]
    iot = lax.broadcasted_iota(jnp.int32, (NQ, nb), 0)
    oh = (iot == r).astype(jnp.float32)
    qre = qre_ref[...]
    dn = (((0,), (0,)), ((), ()))
    f32 = jnp.float32
    g1b = g1b_ref[...]
    qg1 = qre @ g1m_ref[...]
    g = (lax.dot_general(hrb, g1b[:HID], (((1,), (0,)), ((), ())),
                         preferred_element_type=f32) +
         lax.dot_general(hsb, g1b[2 * HID:], (((1,), (0,)), ((), ())),
                         preferred_element_type=f32) +
         lax.dot_general(oh, qg1, dn, preferred_element_type=f32))
    g = 1.0 / (1.0 + jnp.exp(-(g + gb1_ref[...])))
    u = g[:, :HID]
    rs = g[:, HID:]
    g2b = g2b_ref[...]
    rh = (rs * hs).astype(jnp.bfloat16)
    cand = jnp.tanh(
        lax.dot_general(hrb, g2b[:HID], (((1,), (0,)), ((), ())),
                        preferred_element_type=f32) +
        lax.dot_general(rh, g2b[HID:], (((1,), (0,)), ((), ())),
                        preferred_element_type=f32) + gb2_ref[...])
    uri = (1.0 - u) * hs + u * cand
    qa = qre @ Wqr_ref[...] + bqr_ref[...]
    sc = jnp.maximum(uri @ Ws_ref[...] +
                     lax.dot_general(oh, qa, dn,
                                     preferred_element_type=f32), 0.0)
    aw = sc @ wa_ref[...] + ba_ref[...]
    ue = jnp.exp(aw)
    out_ref[...] = jnp.concatenate(
        [ue * uri, ue, jnp.zeros((nb, DM - HID - 1), jnp.float32)], axis=1)


def _full(shape):
    nd = len(shape)
    return pl.BlockSpec(shape, lambda i, _n=nd: (0,) * _n)


@jax.jit
def _tc_edge(hs, hr, r3, qre, g1b, g1m, gb1, g2b, gb2, Ws, Wqr, bqr, wa, ba):
    ne = hs.shape[0]
    nblk = ne // BE
    return pl.pallas_call(
        _edge_body,
        out_shape=jax.ShapeDtypeStruct((ne, DM), jnp.float32),
        grid=(nblk,),
        in_specs=[
            pl.BlockSpec((BE, HID), lambda i: (i, 0)),
            pl.BlockSpec((BE, HID), lambda i: (i, 0)),
            pl.BlockSpec((1, 1, BE), lambda i: (i, 0, 0)),
            _full((NQ, HID)), _full((3 * HID, 2 * HID)),
            _full((HID, 2 * HID)), _full((1, 2 * HID)),
            _full((2 * HID, HID)), _full((1, HID)), _full((HID, ATT)),
            _full((HID, ATT)), _full((1, ATT)), _full((ATT, 1)),
            _full((1, 1)),
        ],
        out_specs=pl.BlockSpec((BE, DM), lambda i: (i, 0)),
    )(hs, hr, r3, qre, g1b, g1m, gb1, g2b, gb2, Ws, Wqr, bqr, wa, ba)


def _node_body(a0_ref, ng_ref, h0_ref, qre_ref, Wh_ref, g1b_ref, g1m_ref,
               gb1_ref, g2b_ref, gb2_ref, Wf_ref, hid_ref, hbf_ref, sc_ref):
    a = a0_ref[...]
    agg = a[:, :HID]
    se = a[:, HID:HID + 1]
    m = jnp.maximum((agg / se) @ Wh_ref[...], 0.0)
    ng = ng_ref[0]
    iot = lax.broadcasted_iota(jnp.int32, (NQ, BN), 0)
    oh = (iot == ng).astype(jnp.float32)
    dn = (((0,), (0,)), ((), ()))
    f32 = jnp.float32
    h = h0_ref[...]
    g1 = g1b_ref[...]
    qg1 = qre_ref[...] @ g1m_ref[...]
    g = (m @ g1[:HID] + h @ g1[2 * HID:] +
         lax.dot_general(oh, qg1, dn, preferred_element_type=f32))
    g = 1.0 / (1.0 + jnp.exp(-(g + gb1_ref[...])))
    u = g[:, :HID]
    rs = g[:, HID:]
    g2 = g2b_ref[...]
    cand = jnp.tanh(m @ g2[:HID] + (rs * h) @ g2[HID:] + gb2_ref[...])
    out = (1.0 - u) * h + u * cand
    hid_ref[...] = out
    hbf_ref[...] = out.astype(jnp.bfloat16)
    sc_ref[...] = out @ Wf_ref[...]


@jax.jit
def _tc_node(a0, ng3, h0, qre, Wh, g1b, g1m, gb1, g2b, gb2, Wf):
    nblk = NNODE // BN
    return pl.pallas_call(
        _node_body,
        out_shape=[jax.ShapeDtypeStruct((NNODE, HID), jnp.float32),
                   jax.ShapeDtypeStruct((NNODE, HID), jnp.bfloat16),
                   jax.ShapeDtypeStruct((NNODE, 1), jnp.float32)],
        grid=(nblk,),
        in_specs=[
            pl.BlockSpec((BN, DM), lambda i: (i, 0)),
            pl.BlockSpec((1, 1, BN), lambda i: (i, 0, 0)),
            pl.BlockSpec((BN, HID), lambda i: (i, 0)),
            _full((NQ, HID)), _full((HID, HID)), _full((3 * HID, 2 * HID)),
            _full((HID, 2 * HID)), _full((1, 2 * HID)),
            _full((2 * HID, HID)), _full((1, HID)),
            _full((HID, 1)),
        ],
        out_specs=[pl.BlockSpec((BN, HID), lambda i: (i, 0)),
                   pl.BlockSpec((BN, HID), lambda i: (i, 0)),
                   pl.BlockSpec((BN, 1), lambda i: (i, 0))],
    )(a0, ng3, h0, qre, Wh, g1b, g1m, gb1, g2b, gb2, Wf)


def kernel(subs, rels, edges, nodes, old_idx, params):
    n = subs.shape[0]
    idt = edges.dtype
    r_idx = edges[:, 0]
    rel = edges[:, 2]
    sub = edges[:, 4]
    obj = edges[:, 5]
    sub2 = old_idx[sub]
    key = rel * NQ + r_idx

    node_group = jnp.zeros((NNODE,), dtype=idt).at[obj].set(r_idx)

    pad = NEP - NE

    def p32(x, fill):
        return jnp.concatenate(
            [x, jnp.full((pad,), fill, x.dtype)]).reshape(NW, K, CH)

    sub_sc = p32(sub, 0)
    sub2_sc = p32(sub2, 0)
    rel_sc = p32(rel, 0)
    obj_sc = p32(obj, NNODE)
    key_sc = p32(key, 0)
    r3 = jnp.concatenate([r_idx, jnp.zeros((pad,), idt)]).reshape(
        NEP // BE, 1, BE)
    ng3 = node_group.reshape(NNODE // BN, 1, BN)
    zrows = jnp.zeros((NSEG // NS, DMH), jnp.float32)

    layers = params["layers"]
    top = params["top"]
    bf = jnp.bfloat16
    tp = [top["gW1"], top["gW1"][HID:2 * HID],
          top["gb1"].reshape(1, -1), top["gW2"],
          top["gb2"].reshape(1, -1)]
    Wf = params["Wfinal"]

    def edge_call(p, hs, hr, r3v, qre):
        return _tc_edge(hs, hr, r3v, qre, p["gW1"].astype(bf),
                        p["gW1"][HID:2 * HID], p["gb1"].reshape(1, -1),
                        p["gW2"].astype(bf), p["gb2"].reshape(1, -1),
                        p["Ws"], p["Wqr"], p["bqr"].reshape(1, -1), p["wa"],
                        p["ba"].reshape(1, -1))

    def node_call(p, aggx, h0_in, qre):
        return _tc_node(aggx[:NNODE], ng3, h0_in, qre, p["Wh"],
                        tp[0], tp[1], tp[2], tp[3], tp[4], Wf)

    qres = [p["rela"][rels] for p in layers]

    hr_tabs = [layers[li]["rela"].astype(bf) for li in range(1, NL + NXL)]
    hr_outs = _sc_gather_multi(tuple(hr_tabs), rel_sc)
    hr_l = [None] + list(hr_outs)

    p0 = layers[0]
    hs0 = jnp.zeros((KEYP, HID), jnp.bfloat16)
    hr0 = jnp.concatenate(
        [jnp.repeat(p0["rela"], NQ, axis=0),
         jnp.zeros((KEYP - NKEY, HID), jnp.float32)]).astype(jnp.bfloat16)
    r0 = jnp.tile(jnp.arange(NQ, dtype=idt), KEYP // NQ).reshape(
        KEYP // BE, 1, BE)
    m0 = edge_call(p0, hs0, hr0, r0, qres[0])
    m0s = jnp.stack([m0[:, :DMH], m0[:, DMH:]])
    aggx = _sc_l0(m0s, key_sc, obj_sc, zrows)
    zeros_h = jnp.zeros((NNODE, HID), jnp.float32)
    hidden, hidden_bf, _ = node_call(p0, aggx, zeros_h, qres[0])
    h0 = hidden
    h0_in = jnp.zeros((NNODE, HID), jnp.float32).at[old_idx].set(h0)

    for li in range(1, NL + NXL):
        p = layers[li]
        sub_idx = sub_sc if li < NL else sub2_sc
        hs = _sc_gather(hidden_bf, sub_idx)
        msgx = edge_call(p, hs, hr_l[li], r3, qres[li])
        aggx = _sc_scatter(msgx, obj_sc, zrows)
        hidden, hidden_bf, sc = node_call(p, aggx, h0_in, qres[li])
        h0_in = hidden

    scores = sc[:, 0]
    scores_all = jnp.zeros((n, NNODE), jnp.float32).at[
        nodes[:, 0], nodes[:, 1]].set(scores)
    return scores_all
